# baseline (speedup 1.0000x reference)
import functools

import jax
import jax.numpy as jnp
from jax import lax
from jax.experimental import pallas as pl
from jax.experimental.pallas import tpu as pltpu
from jax.experimental.pallas import tpu_sc as plsc

EMBED = 256
GROUPS = 8
LEVELS = 4
CAMS = 6
A = 900
SHAPES = ((64, 176), (32, 88), (16, 44), (8, 22))
LEVEL_STARTS = (0, 11264, 14080, 14784)
TOTAL = 14960
CL = CAMS * LEVELS
NR = CL * 4
NW = 32
NPW = 29
APAD = NW * NPW


def _prep_body(inst_ref, emb_ref, anc4_ref, kx_ref, ky_ref, kz_ref, wh_ref,
               wfct_ref, bfc_ref, msum_ref, mexp_ref,
               w8_ref, idx_ref, bil_ref):
    feat = inst_ref[...] + emb_ref[...]
    logits = jnp.dot(feat, wfct_ref[...], preferred_element_type=jnp.float32)
    logits = logits + bfc_ref[...]
    e = jnp.exp(logits)
    s = jnp.dot(e, msum_ref[...], preferred_element_type=jnp.float32)
    den = jnp.dot(s, mexp_ref[...], preferred_element_type=jnp.float32)
    w8_ref[...] = e / den

    anc4 = anc4_ref[...]
    X = jnp.dot(anc4, kx_ref[...], preferred_element_type=jnp.float32)
    Y = jnp.dot(anc4, ky_ref[...], preferred_element_type=jnp.float32)
    Z = jnp.dot(anc4, kz_ref[...], preferred_element_type=jnp.float32)
    z = jnp.maximum(Z, 1e-5)
    whx = wh_ref[0, 0]
    why = wh_ref[0, 1]
    xn = X / (z * whx)
    yn = Y / (z * why)
    camoff = lax.broadcasted_iota(jnp.int32, (APAD, CAMS), 1) * TOTAL

    for l in range(LEVELS):
        Hl, Wl = SHAPES[l]
        start_l = LEVEL_STARTS[l]
        px = jnp.clip(xn * float(Wl) - 0.5, -3.0, float(Wl) + 2.0)
        py = jnp.clip(yn * float(Hl) - 0.5, -3.0, float(Hl) + 2.0)
        x0 = jnp.floor(px)
        y0 = jnp.floor(py)
        fx = px - x0
        fy = py - y0
        x0i = x0.astype(jnp.int32)
        y0i = y0.astype(jnp.int32)
        for dy in (0, 1):
            for dx in (0, 1):
                xi = x0i + dx
                yi = y0i + dy
                valid = (xi >= 0) & (xi < Wl) & (yi >= 0) & (yi < Hl)
                wxy = (fx if dx else 1.0 - fx) * (fy if dy else 1.0 - fy)
                bilv = jnp.where(valid, wxy, 0.0)
                xc = jnp.clip(xi, 0, Wl - 1)
                yc = jnp.clip(yi, 0, Hl - 1)
                rows = camoff + (start_l + yc * Wl + xc)
                corner = dy * 2 + dx
                for c in range(CAMS):
                    col = c * 16 + l * 4 + corner
                    idx_ref[:, col:col + 1] = rows[:, c:c + 1]
                    bil_ref[:, col:col + 1] = bilv[:, c:c + 1]


_prep = pl.pallas_call(
    _prep_body,
    out_shape=[
        jax.ShapeDtypeStruct((APAD, CL * GROUPS), jnp.float32),
        jax.ShapeDtypeStruct((APAD, NR), jnp.int32),
        jax.ShapeDtypeStruct((APAD, NR), jnp.float32),
    ],
)


def _sc_agg_body(idx_hbm, wcl_hbm, feat_hbm, out_hbm,
                 idx_v, wcl_v, rows0, rows1, rows2, rows3, out_v,
                 sem0, sem1, sem2, sem3):
    wid = lax.axis_index("s") * 2 + lax.axis_index("c")
    base = wid * NPW
    pltpu.sync_copy(idx_hbm.at[pl.ds(base * 48, NPW * 48)], idx_v)
    pltpu.sync_copy(wcl_hbm.at[pl.ds(base * CL * 16, NPW * CL * 16)], wcl_v)

    bufs = ((rows0, sem0), (rows1, sem1), (rows2, sem2), (rows3, sem3))

    def _start(k, b):
        r, s = bufs[b]
        pltpu.make_async_copy(
            feat_hbm.at[idx_v.at[pl.ds(k * 48, 48)]], r, s).start()

    def _wait(k, b):
        r, s = bufs[b]
        pltpu.make_async_copy(
            feat_hbm.at[idx_v.at[pl.ds(k * 48, 48)]], r, s).wait()

    def _compute(k, rows_ref):
        for j in range(16):
            out_v[pl.ds(k * EMBED + j * 16, 16)] = rows_ref[0, pl.ds(j * 16, 16)]
        return

        def cl_body(cl, accs):
            r0 = 4 * cl
            wv = wcl_v[pl.ds(k * CL * 16 + cl * 16, 16)]
            b0 = wv[0]
            b1 = wv[1]
            b2 = wv[2]
            b3 = wv[3]
            out = []
            for j in range(16):
                s = pl.ds(j * 16, 16)
                t = (rows_ref[r0, s] * b0 + rows_ref[r0 + 1, s] * b1
                     + rows_ref[r0 + 2, s] * b2 + rows_ref[r0 + 3, s] * b3)
                out.append(accs[j] + t * wv[4 + j // 2])
            return tuple(out)

        accs = lax.fori_loop(
            0, CL, cl_body,
            tuple(jnp.zeros((16,), jnp.float32) for _ in range(16)))
        for j in range(16):
            out_v[pl.ds(k * EMBED + j * 16, 16)] = accs[j]

    for b in range(4):
        _start(b, b)

    def quad(i, carry):
        k0 = 4 * i
        for b in range(4):
            _wait(k0 + b, b)
            _compute(k0 + b, bufs[b][0])
            _start(k0 + b + 4, b)
        return carry

    lax.fori_loop(0, (NPW - 5) // 4, quad, 0)
    for k in range(NPW - 5, NPW):
        b = k % 4
        _wait(k, b)
        _compute(k, bufs[b][0])
        if k + 4 < NPW:
            _start(k + 4, b)

    pltpu.sync_copy(out_v, out_hbm.at[pl.ds(base * EMBED, NPW * EMBED)])


@functools.cache
def _get_sc_agg():
    mesh = plsc.VectorSubcoreMesh(core_axis_name="c", subcore_axis_name="s")
    return pl.kernel(
        _sc_agg_body,
        mesh=mesh,
        out_type=jax.ShapeDtypeStruct((APAD * EMBED,), jnp.float32),
        scratch_types=[
            pltpu.VMEM((NPW * 48,), jnp.int32),
            pltpu.VMEM((NPW * CL * 16,), jnp.float32),
            pltpu.VMEM((48, 512), jnp.float32),
            pltpu.VMEM((48, 512), jnp.float32),
            pltpu.VMEM((48, 512), jnp.float32),
            pltpu.VMEM((48, 512), jnp.float32),
            pltpu.VMEM((NPW * EMBED,), jnp.float32),
            pltpu.SemaphoreType.DMA,
            pltpu.SemaphoreType.DMA,
            pltpu.SemaphoreType.DMA,
            pltpu.SemaphoreType.DMA,
        ],
    )


def _post_body(agg_ref, inst_ref, woutt_ref, bout_ref, o_ref):
    o_ref[...] = (jnp.dot(agg_ref[...], woutt_ref[...],
                          preferred_element_type=jnp.float32)
                  + bout_ref[...] + inst_ref[...])


_post = pl.pallas_call(
    _post_body,
    out_shape=jax.ShapeDtypeStruct((APAD, EMBED), jnp.float32),
)


def kernel(instance_feature, anchor, anchor_embed, feature_flat, spatial_shape,
           level_start_index, projection_mat, image_wh, W_fc, b_fc, W_out,
           b_out):
    pad = APAD - A
    inst = instance_feature[0]
    inst_p = jnp.pad(inst, ((0, pad), (0, 0)))
    emb_p = jnp.pad(anchor_embed[0], ((0, pad), (0, 0)))
    anc4 = jnp.concatenate(
        [anchor[0], jnp.ones((A, 1), jnp.float32)], axis=1)
    anc4_p = jnp.pad(anc4, ((0, pad), (0, 0)))
    proj = projection_mat[0]
    kx = jnp.transpose(proj[:, 0, :])
    ky = jnp.transpose(proj[:, 1, :])
    kz = jnp.transpose(proj[:, 2, :])
    wh2 = image_wh.reshape(-1, 2)[0].reshape(1, 2)
    wfct = jnp.transpose(W_fc)
    bfc = b_fc.reshape(1, -1)
    msum = jnp.tile(jnp.eye(GROUPS, dtype=jnp.float32), (CL, 1))
    mexp = jnp.transpose(msum)

    w8, idx, bil = _prep(inst_p, emb_p, anc4_p, kx, ky, kz, wh2,
                         wfct, bfc, msum, mexp)

    wcl = jnp.concatenate(
        [bil.reshape(APAD, CL, 4), w8.reshape(APAD, CL, GROUPS),
         jnp.zeros((APAD, CL, 4), jnp.float32)], axis=2).reshape(APAD, CL * 16)

    feat2d = feature_flat.reshape(CAMS * TOTAL // 2, EMBED * 2)
    idx48 = (idx.reshape(APAD, 48, 2)[:, :, 0] >> 1)
    agg = _get_sc_agg()(idx48.reshape(-1), wcl.reshape(-1), feat2d)
    agg = agg.reshape(APAD, EMBED)

    out = _post(agg, inst_p, jnp.transpose(W_out), b_out.reshape(1, -1))
    return out[:A].reshape(1, A, EMBED)


# speedup vs baseline: 1.2660x; 1.2660x over previous
import functools

import jax
import jax.numpy as jnp
from jax import lax
from jax.experimental import pallas as pl
from jax.experimental.pallas import tpu as pltpu
from jax.experimental.pallas import tpu_sc as plsc

EMBED = 256
GROUPS = 8
LEVELS = 4
CAMS = 6
A = 900
SHAPES = ((64, 176), (32, 88), (16, 44), (8, 22))
LEVEL_STARTS = (0, 11264, 14080, 14784)
TOTAL = 14960
CL = CAMS * LEVELS
NR = CL * 4
NW = 32
NPW = 29
APAD = NW * NPW
NPAIR = NPW * 8
NCHUNK = (NPAIR + 15) // 16
PLIST = 272
BATCH = 8


def _prep_body(inst_ref, emb_ref, anc4_ref, kx_ref, ky_ref, kz_ref, wh_ref,
               wfct_ref, bfc_ref, msum_ref, mexp_ref,
               w8_ref, idx_ref, bil_ref, act_ref):
    feat = inst_ref[...] + emb_ref[...]
    logits = jnp.dot(feat, wfct_ref[...], preferred_element_type=jnp.float32)
    logits = logits + bfc_ref[...]
    e = jnp.exp(logits)
    s = jnp.dot(e, msum_ref[...], preferred_element_type=jnp.float32)
    den = jnp.dot(s, mexp_ref[...], preferred_element_type=jnp.float32)
    w8_ref[...] = e / den

    anc4 = anc4_ref[...]
    X = jnp.dot(anc4, kx_ref[...], preferred_element_type=jnp.float32)
    Y = jnp.dot(anc4, ky_ref[...], preferred_element_type=jnp.float32)
    Z = jnp.dot(anc4, kz_ref[...], preferred_element_type=jnp.float32)
    z = jnp.maximum(Z, 1e-5)
    whx = wh_ref[0, 0]
    why = wh_ref[0, 1]
    xn = X / (z * whx)
    yn = Y / (z * why)
    camoff = lax.broadcasted_iota(jnp.int32, (APAD, CAMS), 1) * TOTAL
    rowmask = (lax.broadcasted_iota(jnp.int32, (APAD, 1), 0)
               < A).astype(jnp.float32)

    act_ref[...] = jnp.zeros((APAD + 2, 8), jnp.float32)
    act6 = jnp.zeros((APAD, CAMS), jnp.float32)
    for l in range(LEVELS):
        Hl, Wl = SHAPES[l]
        start_l = LEVEL_STARTS[l]
        px = jnp.clip(xn * float(Wl) - 0.5, -3.0, float(Wl) + 2.0)
        py = jnp.clip(yn * float(Hl) - 0.5, -3.0, float(Hl) + 2.0)
        x0 = jnp.floor(px)
        y0 = jnp.floor(py)
        fx = px - x0
        fy = py - y0
        x0i = x0.astype(jnp.int32)
        y0i = y0.astype(jnp.int32)
        for dy in (0, 1):
            for dx in (0, 1):
                xi = x0i + dx
                yi = y0i + dy
                valid = (xi >= 0) & (xi < Wl) & (yi >= 0) & (yi < Hl)
                wxy = (fx if dx else 1.0 - fx) * (fy if dy else 1.0 - fy)
                bilv = jnp.where(valid, wxy, 0.0) * rowmask
                act6 = act6 + bilv
                xc = jnp.clip(xi, 0, Wl - 1)
                yc = jnp.clip(yi, 0, Hl - 1)
                rows = camoff + (start_l + yc * Wl + xc)
                corner = dy * 2 + dx
                for c in range(CAMS):
                    col = c * 16 + l * 4 + corner
                    idx_ref[:, col:col + 1] = rows[:, c:c + 1]
                    bil_ref[:, col:col + 1] = bilv[:, c:c + 1]
    for c in range(CAMS):
        act_ref[0:APAD, c:c + 1] = act6[:, c:c + 1]


_prep = pl.pallas_call(
    _prep_body,
    out_shape=[
        jax.ShapeDtypeStruct((APAD, CL * GROUPS), jnp.float32),
        jax.ShapeDtypeStruct((APAD, NR), jnp.int32),
        jax.ShapeDtypeStruct((APAD, NR), jnp.float32),
        jax.ShapeDtypeStruct((APAD + 2, 8), jnp.float32),
    ],
)


def _sc_agg_body(idx_hbm, wcl_hbm, act_hbm, feat_hbm, out_hbm,
                 idx_v, wcl_v, act_v, plist_sh, plist, stage, g0, g1,
                 rows0, rows1, out_v, sem0, sem1):
    sid = lax.axis_index("s")
    wid = sid * 2 + lax.axis_index("c")
    base = wid * NPW
    pltpu.sync_copy(idx_hbm.at[pl.ds(base * NR, NPW * NR)], idx_v)
    pltpu.sync_copy(wcl_hbm.at[pl.ds(base * CL * 16, NPW * CL * 16)], wcl_v)
    pltpu.sync_copy(act_hbm.at[pl.ds(base * 8, NCHUNK * 16)], act_v)

    zeros16i = jnp.zeros((16,), jnp.int32)
    zeros16f = jnp.zeros((16,), jnp.float32)
    for i in range(PLIST // 16):
        plist[pl.ds(i * 16, 16)] = zeros16i
    pltpu.sync_copy(plist, plist_sh.at[pl.ds(sid * PLIST, PLIST)])

    def zero_out(k, carry):
        for j in range(16):
            out_v[pl.ds(k * EMBED + j * 16, 16)] = zeros16f
        return carry

    lax.fori_loop(0, NPW, zero_out, 0)

    lanes = lax.broadcasted_iota(jnp.int32, (16,), 0)

    _gd = lax.GatherDimensionNumbers(
        offset_dims=(), collapsed_slice_dims=(0,), start_index_map=(0,))

    def _prefix16(v):
        for sh in (1, 2, 4, 8):
            src = jnp.maximum(lanes - sh, 0)
            shifted = lax.gather(
                v, src[:, None], _gd, (1,),
                mode=lax.GatherScatterMode.PROMISE_IN_BOUNDS)
            v = v + jnp.where(lanes >= sh, shifted, jnp.int32(0))
        return v

    def scan_chunk(ch, offset):
        flags = act_v[pl.ds(ch * 16, 16)]
        pid = lanes + ch * 16
        mask = (flags > 0.0) & (pid < NPAIR)
        mi = jnp.where(mask, jnp.int32(1), jnp.int32(0))
        cs = _prefix16(mi)
        pos = jnp.where(mask, offset + cs - 1, jnp.int32(PLIST - 1))
        stage[...] = pid
        pltpu.sync_copy(stage, plist_sh.at[sid * PLIST + pos])
        return offset + cs[15]

    n_active = lax.fori_loop(0, NCHUNK, scan_chunk, jnp.int32(0))
    pltpu.sync_copy(plist_sh.at[pl.ds(sid * PLIST, PLIST)], plist)
    nb = (n_active + BATCH - 1) >> 3

    gbufs = ((g0, rows0, sem0), (g1, rows1, sem1))

    def _build(b, gb):
        glist = gbufs[gb][0]
        pv = plist[pl.ds(b * BATCH, 16)]
        for s in range(BATCH):
            p = pv[s]
            q = p >> 3
            c = p & 7
            pairbase = (q * 6 + c) * 16
            glist[pl.ds(s * 16, 16)] = idx_v[pl.ds(pairbase, 16)]

    def _start(gb):
        glist, rows, sem = gbufs[gb]
        pltpu.make_async_copy(feat_hbm.at[glist], rows, sem).start()

    def _wait(gb):
        glist, rows, sem = gbufs[gb]
        pltpu.make_async_copy(feat_hbm.at[glist], rows, sem).wait()

    def _compute(b, gb):
        rows_ref = gbufs[gb][1]
        pv = plist[pl.ds(b * BATCH, 16)]
        for s in range(BATCH):
            p = pv[s]
            q = p >> 3
            c = p & 7
            m = jnp.where(b * BATCH + s < n_active, 1.0, 0.0)
            wbase = (q * 6 + c) * 64
            obase = q * EMBED

            def l_body(l, accs):
                wv = wcl_v[pl.ds(wbase + l * 16, 16)]
                b0 = wv[0] * m
                b1 = wv[1] * m
                b2 = wv[2] * m
                b3 = wv[3] * m
                rbase = s * 16 + l * 4
                nxt = []
                for j in range(16):
                    sl = pl.ds(j * 16, 16)
                    t = (rows_ref[rbase, sl] * b0 + rows_ref[rbase + 1, sl] * b1
                         + rows_ref[rbase + 2, sl] * b2
                         + rows_ref[rbase + 3, sl] * b3)
                    nxt.append(accs[j] + t * wv[4 + j // 2])
                return tuple(nxt)

            accs = lax.fori_loop(0, LEVELS, l_body,
                                 tuple(zeros16f for _ in range(16)))
            for j in range(16):
                plsc.addupdate(out_v.at[pl.ds(obase + j * 16, 16)], accs[j])

    @pl.when(nb >= 1)
    def _():
        _build(0, 0)
        _start(0)

    @pl.when(nb >= 2)
    def _():
        _build(1, 1)
        _start(1)

    def pair_iter(i, carry):
        for gb in range(2):
            b = 2 * i + gb

            @pl.when(b < nb)
            def _():
                _wait(gb)
                _compute(b, gb)

                @pl.when(b + 2 < nb)
                def _():
                    _build(b + 2, gb)
                    _start(gb)
        return carry

    lax.fori_loop(0, (nb + 1) >> 1, pair_iter, 0)

    pltpu.sync_copy(out_v, out_hbm.at[pl.ds(base * EMBED, NPW * EMBED)])


@functools.cache
def _get_sc_agg():
    mesh = plsc.VectorSubcoreMesh(core_axis_name="c", subcore_axis_name="s")
    return pl.kernel(
        _sc_agg_body,
        mesh=mesh,
        out_type=jax.ShapeDtypeStruct((APAD * EMBED,), jnp.float32),
        scratch_types=[
            pltpu.VMEM((NPW * NR,), jnp.int32),
            pltpu.VMEM((NPW * CL * 16,), jnp.float32),
            pltpu.VMEM((NCHUNK * 16,), jnp.float32),
            pltpu.VMEM_SHARED((16 * PLIST,), jnp.int32),
            pltpu.VMEM((PLIST,), jnp.int32),
            pltpu.VMEM((16,), jnp.int32),
            pltpu.VMEM((BATCH * 16,), jnp.int32),
            pltpu.VMEM((BATCH * 16,), jnp.int32),
            pltpu.VMEM((BATCH * 16, EMBED), jnp.float32),
            pltpu.VMEM((BATCH * 16, EMBED), jnp.float32),
            pltpu.VMEM((NPW * EMBED,), jnp.float32),
            pltpu.SemaphoreType.DMA,
            pltpu.SemaphoreType.DMA,
        ],
    )


def _post_body(agg_ref, inst_ref, woutt_ref, bout_ref, o_ref):
    o_ref[...] = (jnp.dot(agg_ref[...], woutt_ref[...],
                          preferred_element_type=jnp.float32)
                  + bout_ref[...] + inst_ref[...])


_post = pl.pallas_call(
    _post_body,
    out_shape=jax.ShapeDtypeStruct((APAD, EMBED), jnp.float32),
)


def kernel(instance_feature, anchor, anchor_embed, feature_flat, spatial_shape,
           level_start_index, projection_mat, image_wh, W_fc, b_fc, W_out,
           b_out):
    pad = APAD - A
    inst = instance_feature[0]
    inst_p = jnp.pad(inst, ((0, pad), (0, 0)))
    emb_p = jnp.pad(anchor_embed[0], ((0, pad), (0, 0)))
    anc4 = jnp.concatenate(
        [anchor[0], jnp.ones((A, 1), jnp.float32)], axis=1)
    anc4_p = jnp.pad(anc4, ((0, pad), (0, 0)))
    proj = projection_mat[0]
    kx = jnp.transpose(proj[:, 0, :])
    ky = jnp.transpose(proj[:, 1, :])
    kz = jnp.transpose(proj[:, 2, :])
    wh2 = image_wh.reshape(-1, 2)[0].reshape(1, 2)
    wfct = jnp.transpose(W_fc)
    bfc = b_fc.reshape(1, -1)
    msum = jnp.tile(jnp.eye(GROUPS, dtype=jnp.float32), (CL, 1))
    mexp = jnp.transpose(msum)

    w8, idx, bil, act = _prep(inst_p, emb_p, anc4_p, kx, ky, kz, wh2,
                              wfct, bfc, msum, mexp)

    wcl = jnp.concatenate(
        [bil.reshape(APAD, CL, 4), w8.reshape(APAD, CL, GROUPS),
         jnp.zeros((APAD, CL, 4), jnp.float32)], axis=2).reshape(APAD, CL * 16)

    feat2d = feature_flat.reshape(CAMS * TOTAL, EMBED)
    agg = _get_sc_agg()(idx.reshape(-1), wcl.reshape(-1),
                        act.reshape(-1), feat2d)
    agg = agg.reshape(APAD, EMBED)

    out = _post(agg, inst_p, jnp.transpose(W_out), b_out.reshape(1, -1))
    return out[:A].reshape(1, A, EMBED)


# speedup vs baseline: 1.2751x; 1.0072x over previous
import functools

import jax
import jax.numpy as jnp
from jax import lax
from jax.experimental import pallas as pl
from jax.experimental.pallas import tpu as pltpu
from jax.experimental.pallas import tpu_sc as plsc

EMBED = 256
GROUPS = 8
LEVELS = 4
CAMS = 6
A = 900
SHAPES = ((64, 176), (32, 88), (16, 44), (8, 22))
LEVEL_STARTS = (0, 11264, 14080, 14784)
TOTAL = 14960
CL = CAMS * LEVELS
NR = CL * 4
NW = 32
NPW = 29
APAD = NW * NPW
NPAIR = NPW * 8
NCHUNK = (NPAIR + 15) // 16
PLIST = 272
BATCH = 8


def _prep_body(inst_ref, emb_ref, anc4_ref, kx_ref, ky_ref, kz_ref, wh_ref,
               wfct_ref, bfc_ref, msum_ref, mexp_ref,
               w8_ref, idx_ref, bil_ref, act_ref):
    feat = inst_ref[...] + emb_ref[...]
    logits = jnp.dot(feat, wfct_ref[...], preferred_element_type=jnp.float32)
    logits = logits + bfc_ref[...]
    e = jnp.exp(logits)
    s = jnp.dot(e, msum_ref[...], preferred_element_type=jnp.float32)
    den = jnp.dot(s, mexp_ref[...], preferred_element_type=jnp.float32)
    w8_ref[...] = e / den

    anc4 = anc4_ref[...]
    X = jnp.dot(anc4, kx_ref[...], preferred_element_type=jnp.float32)
    Y = jnp.dot(anc4, ky_ref[...], preferred_element_type=jnp.float32)
    Z = jnp.dot(anc4, kz_ref[...], preferred_element_type=jnp.float32)
    z = jnp.maximum(Z, 1e-5)
    whx = wh_ref[0, 0]
    why = wh_ref[0, 1]
    xn = X / (z * whx)
    yn = Y / (z * why)
    camoff = lax.broadcasted_iota(jnp.int32, (APAD, CAMS), 1) * TOTAL
    rowmask = (lax.broadcasted_iota(jnp.int32, (APAD, 1), 0)
               < A).astype(jnp.float32)

    act_ref[...] = jnp.zeros((APAD + 2, 8), jnp.float32)
    act6 = jnp.zeros((APAD, CAMS), jnp.float32)
    for l in range(LEVELS):
        Hl, Wl = SHAPES[l]
        start_l = LEVEL_STARTS[l]
        px = jnp.clip(xn * float(Wl) - 0.5, -3.0, float(Wl) + 2.0)
        py = jnp.clip(yn * float(Hl) - 0.5, -3.0, float(Hl) + 2.0)
        x0 = jnp.floor(px)
        y0 = jnp.floor(py)
        fx = px - x0
        fy = py - y0
        x0i = x0.astype(jnp.int32)
        y0i = y0.astype(jnp.int32)
        for dy in (0, 1):
            for dx in (0, 1):
                xi = x0i + dx
                yi = y0i + dy
                valid = (xi >= 0) & (xi < Wl) & (yi >= 0) & (yi < Hl)
                wxy = (fx if dx else 1.0 - fx) * (fy if dy else 1.0 - fy)
                bilv = jnp.where(valid, wxy, 0.0) * rowmask
                act6 = act6 + bilv
                xc = jnp.clip(xi, 0, Wl - 1)
                yc = jnp.clip(yi, 0, Hl - 1)
                rows = camoff + (start_l + yc * Wl + xc)
                corner = dy * 2 + dx
                for c in range(CAMS):
                    col = c * 16 + l * 4 + corner
                    idx_ref[:, col:col + 1] = rows[:, c:c + 1]
                    bil_ref[:, col:col + 1] = bilv[:, c:c + 1]
    for c in range(CAMS):
        act_ref[0:APAD, c:c + 1] = act6[:, c:c + 1]


_prep = pl.pallas_call(
    _prep_body,
    out_shape=[
        jax.ShapeDtypeStruct((APAD, CL * GROUPS), jnp.float32),
        jax.ShapeDtypeStruct((APAD, NR), jnp.int32),
        jax.ShapeDtypeStruct((APAD, NR), jnp.float32),
        jax.ShapeDtypeStruct((APAD + 2, 8), jnp.float32),
    ],
)


def _sc_agg_body(idx_hbm, wcl_hbm, act_hbm, feat_hbm, out_hbm,
                 idx_v, wcl_v, act_v, plist_sh, plist, stage, g0, g1,
                 rows0, rows1, out_v, sem0, sem1, semp, semc):
    sid = lax.axis_index("s")
    wid = sid * 2 + lax.axis_index("c")
    base = wid * NPW
    cp_idx = pltpu.make_async_copy(
        idx_hbm.at[pl.ds(base * NR, NPW * NR)], idx_v, semp)
    cp_wcl = pltpu.make_async_copy(
        wcl_hbm.at[pl.ds(base * CL * 16, NPW * CL * 16)], wcl_v, semp)
    cp_act = pltpu.make_async_copy(
        act_hbm.at[pl.ds(base * 8, NCHUNK * 16)], act_v, semp)
    cp_idx.start()
    cp_wcl.start()
    cp_act.start()

    zeros16i = jnp.zeros((16,), jnp.int32)
    zeros16f = jnp.zeros((16,), jnp.float32)
    for i in range(PLIST // 16):
        plist[pl.ds(i * 16, 16)] = zeros16i
    pltpu.sync_copy(plist, plist_sh.at[pl.ds(sid * PLIST, PLIST)])

    def zero_out(k, carry):
        for j in range(16):
            out_v[pl.ds(k * EMBED + j * 16, 16)] = zeros16f
        return carry

    lax.fori_loop(0, NPW, zero_out, 0)

    lanes = lax.broadcasted_iota(jnp.int32, (16,), 0)

    _gd = lax.GatherDimensionNumbers(
        offset_dims=(), collapsed_slice_dims=(0,), start_index_map=(0,))

    def _prefix16(v):
        for sh in (1, 2, 4, 8):
            src = jnp.maximum(lanes - sh, 0)
            shifted = lax.gather(
                v, src[:, None], _gd, (1,),
                mode=lax.GatherScatterMode.PROMISE_IN_BOUNDS)
            v = v + jnp.where(lanes >= sh, shifted, jnp.int32(0))
        return v

    def scan_chunk(ch, offset):
        flags = act_v[pl.ds(ch * 16, 16)]
        pid = lanes + ch * 16
        mask = (flags > 0.0) & (pid < NPAIR)
        mi = jnp.where(mask, jnp.int32(1), jnp.int32(0))
        cs = _prefix16(mi)
        pos = jnp.where(mask, offset + cs - 1, jnp.int32(PLIST - 1))
        stage[pl.ds(ch * 16, 16)] = pid
        pltpu.make_async_copy(
            stage.at[pl.ds(ch * 16, 16)],
            plist_sh.at[sid * PLIST + pos], semc).start()
        return offset + cs[15]

    cp_idx.wait()
    cp_wcl.wait()
    cp_act.wait()
    n_active = lax.fori_loop(0, NCHUNK, scan_chunk, jnp.int32(0))

    def drain_chunk(ch, carry):
        pltpu.make_async_copy(
            stage.at[pl.ds(ch * 16, 16)],
            plist_sh.at[sid * PLIST + lanes], semc).wait()
        return carry

    lax.fori_loop(0, NCHUNK, drain_chunk, 0)
    pltpu.sync_copy(plist_sh.at[pl.ds(sid * PLIST, PLIST)], plist)
    nb = (n_active + BATCH - 1) >> 3

    gbufs = ((g0, rows0, sem0), (g1, rows1, sem1))

    def _build(b, gb):
        glist = gbufs[gb][0]
        pv = plist[pl.ds(b * BATCH, 16)]
        for s in range(BATCH):
            p = pv[s]
            q = p >> 3
            c = p & 7
            pairbase = (q * 6 + c) * 16
            glist[pl.ds(s * 16, 16)] = idx_v[pl.ds(pairbase, 16)]

    def _start(gb):
        glist, rows, sem = gbufs[gb]
        pltpu.make_async_copy(feat_hbm.at[glist], rows, sem).start()

    def _wait(gb):
        glist, rows, sem = gbufs[gb]
        pltpu.make_async_copy(feat_hbm.at[glist], rows, sem).wait()

    def _compute(b, gb):
        rows_ref = gbufs[gb][1]
        pv = plist[pl.ds(b * BATCH, 16)]
        for s in range(BATCH):
            p = pv[s]
            q = p >> 3
            c = p & 7
            m = jnp.where(b * BATCH + s < n_active, 1.0, 0.0)
            wbase = (q * 6 + c) * 64
            obase = q * EMBED

            def l_body(l, accs):
                wv = wcl_v[pl.ds(wbase + l * 16, 16)]
                b0 = wv[0] * m
                b1 = wv[1] * m
                b2 = wv[2] * m
                b3 = wv[3] * m
                rbase = s * 16 + l * 4
                nxt = []
                for j in range(16):
                    sl = pl.ds(j * 16, 16)
                    t = (rows_ref[rbase, sl] * b0 + rows_ref[rbase + 1, sl] * b1
                         + rows_ref[rbase + 2, sl] * b2
                         + rows_ref[rbase + 3, sl] * b3)
                    nxt.append(accs[j] + t * wv[4 + j // 2])
                return tuple(nxt)

            accs = lax.fori_loop(0, LEVELS, l_body,
                                 tuple(zeros16f for _ in range(16)))
            for j in range(16):
                plsc.addupdate(out_v.at[pl.ds(obase + j * 16, 16)], accs[j])

    @pl.when(nb >= 1)
    def _():
        _build(0, 0)
        _start(0)

    @pl.when(nb >= 2)
    def _():
        _build(1, 1)
        _start(1)

    def pair_iter(i, carry):
        for gb in range(2):
            b = 2 * i + gb

            @pl.when(b < nb)
            def _():
                _wait(gb)
                _compute(b, gb)

                @pl.when(b + 2 < nb)
                def _():
                    _build(b + 2, gb)
                    _start(gb)
        return carry

    lax.fori_loop(0, (nb + 1) >> 1, pair_iter, 0)

    pltpu.sync_copy(out_v, out_hbm.at[pl.ds(base * EMBED, NPW * EMBED)])


@functools.cache
def _get_sc_agg():
    mesh = plsc.VectorSubcoreMesh(core_axis_name="c", subcore_axis_name="s")
    return pl.kernel(
        _sc_agg_body,
        mesh=mesh,
        out_type=jax.ShapeDtypeStruct((APAD * EMBED,), jnp.float32),
        scratch_types=[
            pltpu.VMEM((NPW * NR,), jnp.int32),
            pltpu.VMEM((NPW * CL * 16,), jnp.float32),
            pltpu.VMEM((NCHUNK * 16,), jnp.float32),
            pltpu.VMEM_SHARED((16 * PLIST,), jnp.int32),
            pltpu.VMEM((PLIST,), jnp.int32),
            pltpu.VMEM((NCHUNK * 16,), jnp.int32),
            pltpu.VMEM((BATCH * 16,), jnp.int32),
            pltpu.VMEM((BATCH * 16,), jnp.int32),
            pltpu.VMEM((BATCH * 16, EMBED), jnp.float32),
            pltpu.VMEM((BATCH * 16, EMBED), jnp.float32),
            pltpu.VMEM((NPW * EMBED,), jnp.float32),
            pltpu.SemaphoreType.DMA,
            pltpu.SemaphoreType.DMA,
            pltpu.SemaphoreType.DMA,
            pltpu.SemaphoreType.DMA,
        ],
    )


def _post_body(agg_ref, inst_ref, woutt_ref, bout_ref, o_ref):
    o_ref[...] = (jnp.dot(agg_ref[...], woutt_ref[...],
                          preferred_element_type=jnp.float32)
                  + bout_ref[...] + inst_ref[...])


_post = pl.pallas_call(
    _post_body,
    out_shape=jax.ShapeDtypeStruct((APAD, EMBED), jnp.float32),
)


def kernel(instance_feature, anchor, anchor_embed, feature_flat, spatial_shape,
           level_start_index, projection_mat, image_wh, W_fc, b_fc, W_out,
           b_out):
    pad = APAD - A
    inst = instance_feature[0]
    inst_p = jnp.pad(inst, ((0, pad), (0, 0)))
    emb_p = jnp.pad(anchor_embed[0], ((0, pad), (0, 0)))
    anc4 = jnp.concatenate(
        [anchor[0], jnp.ones((A, 1), jnp.float32)], axis=1)
    anc4_p = jnp.pad(anc4, ((0, pad), (0, 0)))
    proj = projection_mat[0]
    kx = jnp.transpose(proj[:, 0, :])
    ky = jnp.transpose(proj[:, 1, :])
    kz = jnp.transpose(proj[:, 2, :])
    wh2 = image_wh.reshape(-1, 2)[0].reshape(1, 2)
    wfct = jnp.transpose(W_fc)
    bfc = b_fc.reshape(1, -1)
    msum = jnp.tile(jnp.eye(GROUPS, dtype=jnp.float32), (CL, 1))
    mexp = jnp.transpose(msum)

    w8, idx, bil, act = _prep(inst_p, emb_p, anc4_p, kx, ky, kz, wh2,
                              wfct, bfc, msum, mexp)

    wcl = jnp.concatenate(
        [bil.reshape(APAD, CL, 4), w8.reshape(APAD, CL, GROUPS),
         jnp.zeros((APAD, CL, 4), jnp.float32)], axis=2).reshape(APAD, CL * 16)

    feat2d = feature_flat.reshape(CAMS * TOTAL, EMBED)
    agg = _get_sc_agg()(idx.reshape(-1), wcl.reshape(-1),
                        act.reshape(-1), feat2d)
    agg = agg.reshape(APAD, EMBED)

    out = _post(agg, inst_p, jnp.transpose(W_out), b_out.reshape(1, -1))
    return out[:A].reshape(1, A, EMBED)
